# baseline (speedup 1.0000x reference)
import functools

import jax
import jax.numpy as jnp
from jax.experimental import pallas as pl
from jax.experimental.pallas import tpu as pltpu

N_G = 1024
N_LG = 4096
F = 128
NB = 32
BM = N_LG // NB
XB = N_G // NB
HALF = F // 2
EPS = 1e-5


def _dot_t(z, w_ref):
    return jax.lax.dot_general(
        z, w_ref[...].astype(jnp.bfloat16),
        (((1,), (1,)), ((), ())), preferred_element_type=jnp.float32)


def _bn(z, s_ref, b_ref):
    m = jnp.mean(z, axis=0, keepdims=True)
    v = jnp.mean((z - m) ** 2, axis=0, keepdims=True)
    return (z - m) * jax.lax.rsqrt(v + EPS) * s_ref[...] + b_ref[...]


def _relu_hi(z):
    col = jax.lax.broadcasted_iota(jnp.int32, z.shape, 1)
    return jnp.where(col < HALF, z, jnp.maximum(z, 0.0))


def _body(mlt_ref, mltt_ref, mgt_ref, mgtt_ref, x_ref, y_ref,
          deg_g_ref, deg_lg_ref, pm_ref, g_ref,
          wtx_ref, wtd_ref, wty_ref, wtl0_ref, wtl1_ref,
          wgy_ref, wgd_ref, wgx_ref, wgl0_ref, wgl1_ref,
          bias_x_ref, bias_y_ref,
          bnx_s_ref, bnx_b_ref, bny_s_ref, bny_b_ref,
          xn_ref, yn_ref,
          ybf_ref, xbf_ref, xpre_ref, acc_ref, ys1_ref, ys2_ref):
    i = pl.program_id(0)

    @pl.when(i == 0)
    def _init():
        ybf_ref[...] = y_ref[...].astype(jnp.bfloat16)
        xbf_ref[...] = x_ref[...].astype(jnp.bfloat16)
        acc_ref[...] = jnp.zeros_like(acc_ref)
        ys1_ref[...] = jnp.zeros_like(ys1_ref)
        ys2_ref[...] = jnp.zeros_like(ys2_ref)

    ybf = ybf_ref[...]
    xbf = xbf_ref[...]

    y0 = jnp.dot(mlt_ref[...].astype(jnp.bfloat16), ybf,
                 preferred_element_type=jnp.float32)
    y1 = jnp.dot(mltt_ref[...].astype(jnp.bfloat16), ybf,
                 preferred_element_type=jnp.float32)
    y_rows = y_ref[pl.ds(i * BM, BM), :]
    lane = jax.lax.broadcasted_iota(jnp.int32, (BM, N_G), 1)
    oh_pm = (lane == pm_ref[pl.ds(i * BM, BM), :]).astype(jnp.bfloat16)
    oh_g = (lane == g_ref[pl.ds(i * BM, BM), :]).astype(jnp.bfloat16)
    pmx = jnp.dot(oh_pm, xbf, preferred_element_type=jnp.float32)
    yn_rows = (_dot_t(y0.astype(jnp.bfloat16), wgl0_ref)
               + _dot_t(y1.astype(jnp.bfloat16), wgl1_ref)
               + _dot_t(y_rows.astype(jnp.bfloat16), wgy_ref)
               + _dot_t((y_rows * deg_lg_ref[pl.ds(i * BM, BM), :])
                        .astype(jnp.bfloat16), wgd_ref)
               + _dot_t(pmx.astype(jnp.bfloat16), wgx_ref)
               + bias_y_ref[...])
    yn_rows = _relu_hi(yn_rows)
    yn_ref[pl.ds(i * BM, BM), :] = yn_rows
    ys1_ref[...] += jnp.sum(yn_rows, axis=0, keepdims=True)
    ys2_ref[...] += jnp.sum(yn_rows * yn_rows, axis=0, keepdims=True)

    x0 = jnp.dot(mgt_ref[...].astype(jnp.bfloat16), xbf,
                 preferred_element_type=jnp.float32)
    x1 = jnp.dot(mgtt_ref[...].astype(jnp.bfloat16), xbf,
                 preferred_element_type=jnp.float32)
    x_rows = x_ref[pl.ds(i * XB, XB), :]
    xpre_ref[pl.ds(i * XB, XB), :] = (
        _dot_t(x0.astype(jnp.bfloat16), wtl0_ref)
        + _dot_t(x1.astype(jnp.bfloat16), wtl1_ref)
        + _dot_t(x_rows.astype(jnp.bfloat16), wtx_ref)
        + _dot_t((x_rows * deg_g_ref[pl.ds(i * XB, XB), :])
                 .astype(jnp.bfloat16), wtd_ref)
        + bias_x_ref[...])

    y_blk_bf = ybf_ref[pl.ds(i * BM, BM), :]
    acc_ref[...] += jax.lax.dot_general(
        oh_g, y_blk_bf, (((0,), (0,)), ((), ())),
        preferred_element_type=jnp.float32)

    @pl.when(i == NB - 1)
    def _finish():
        xn_pre = xpre_ref[...] + _dot_t(acc_ref[...].astype(jnp.bfloat16),
                                        wty_ref)
        xn_ref[...] = _bn(_relu_hi(xn_pre), bnx_s_ref, bnx_b_ref)
        m = ys1_ref[...] * (1.0 / N_LG)
        v = ys2_ref[...] * (1.0 / N_LG) - m * m
        scale = jax.lax.rsqrt(v + EPS) * bny_s_ref[...]
        yn_ref[...] = (yn_ref[...] - m) * scale + bny_b_ref[...]


@functools.partial(jax.jit, static_argnames=("interpret",))
def _run(x, y, deg_g, deg_lg, pm_pd2, g2,
         mask_g_t, mask_g_tt, mask_lg_t, mask_lg_tt,
         Wtx, Wtd, Wty, Wtl0, Wtl1, Wgy, Wgd, Wgx, Wgl0, Wgl1,
         bias_x, bias_y, bnx_s, bnx_b, bny_s, bny_b, interpret=False):
    const = lambda i: (0, 0)
    row_lg = lambda i: (i, 0)
    wspec = pl.BlockSpec((F, F), const)
    vspec = pl.BlockSpec((1, F), const)
    return pl.pallas_call(
        _body,
        grid=(NB,),
        in_specs=[
            pl.BlockSpec((BM, N_LG), row_lg),
            pl.BlockSpec((BM, N_LG), row_lg),
            pl.BlockSpec((XB, N_G), row_lg),
            pl.BlockSpec((XB, N_G), row_lg),
            pl.BlockSpec((N_G, F), const),
            pl.BlockSpec((N_LG, F), const),
            pl.BlockSpec((N_G, 1), const),
            pl.BlockSpec((N_LG, 1), const),
            pl.BlockSpec((N_LG, 1), const),
            pl.BlockSpec((N_LG, 1), const),
            wspec, wspec, wspec, wspec, wspec,
            wspec, wspec, wspec, wspec, wspec,
            vspec, vspec,
            vspec, vspec, vspec, vspec,
        ],
        out_specs=(pl.BlockSpec((N_G, F), const),
                   pl.BlockSpec((N_LG, F), const)),
        out_shape=(jax.ShapeDtypeStruct((N_G, F), jnp.float32),
                   jax.ShapeDtypeStruct((N_LG, F), jnp.float32)),
        scratch_shapes=[
            pltpu.VMEM((N_LG, F), jnp.bfloat16),
            pltpu.VMEM((N_G, F), jnp.bfloat16),
            pltpu.VMEM((N_G, F), jnp.float32),
            pltpu.VMEM((N_G, F), jnp.float32),
            pltpu.VMEM((1, F), jnp.float32),
            pltpu.VMEM((1, F), jnp.float32),
        ],
        compiler_params=pltpu.CompilerParams(
            dimension_semantics=("arbitrary",),
        ),
        interpret=interpret,
    )(mask_lg_t, mask_lg_tt, mask_g_t, mask_g_tt, x, y,
      deg_g, deg_lg, pm_pd2, g2,
      Wtx, Wtd, Wty, Wtl0, Wtl1, Wgy, Wgd, Wgx, Wgl0, Wgl1,
      bias_x, bias_y, bnx_s, bnx_b, bny_s, bny_b)


def kernel(g, lg, x, y, deg_g, deg_lg, pm_pd, g_t, g_tt, lg_t, lg_tt,
           mask_g_t, mask_g_tt, mask_lg_t, mask_lg_tt,
           Wtx, btx, Wtd, btd, Wty, bty, Wtl0, btl0, Wtl1, btl1,
           Wgy, bgy, Wgd, bgd, Wgx, bgx, Wgl0, bgl0, Wgl1, bgl1,
           bnx_s, bnx_b, bny_s, bny_b):
    bias_x = (btx + btd + btl0 + btl1 + bty).reshape(1, F)
    bias_y = (bgy + bgd + bgl0 + bgl1 + bgx).reshape(1, F)
    return _run(x, y, deg_g, deg_lg,
                pm_pd.astype(jnp.int32).reshape(N_LG, 1),
                g.astype(jnp.int32).reshape(N_LG, 1),
                mask_g_t, mask_g_tt, mask_lg_t, mask_lg_tt,
                Wtx, Wtd, Wty, Wtl0, Wtl1, Wgy, Wgd, Wgx, Wgl0, Wgl1,
                bias_x, bias_y,
                bnx_s.reshape(1, F), bnx_b.reshape(1, F),
                bny_s.reshape(1, F), bny_b.reshape(1, F))


# speedup vs baseline: 1.1659x; 1.1659x over previous
import functools

import jax
import jax.numpy as jnp
from jax.experimental import pallas as pl
from jax.experimental.pallas import tpu as pltpu

N_G = 1024
N_LG = 4096
F = 128
NB = 16
BM = N_LG // NB
XB = N_G // NB
HALF = F // 2
EPS = 1e-5


def _dot_t(z, w_ref):
    return jax.lax.dot_general(
        z, w_ref[...].astype(jnp.bfloat16),
        (((1,), (1,)), ((), ())), preferred_element_type=jnp.float32)


def _bn(z, s_ref, b_ref):
    m = jnp.mean(z, axis=0, keepdims=True)
    v = jnp.mean((z - m) ** 2, axis=0, keepdims=True)
    return (z - m) * jax.lax.rsqrt(v + EPS) * s_ref[...] + b_ref[...]


def _relu_hi(z):
    col = jax.lax.broadcasted_iota(jnp.int32, z.shape, 1)
    return jnp.where(col < HALF, z, jnp.maximum(z, 0.0))


def _body(mlt_ref, mltt_ref, mgt_ref, mgtt_ref, x_ref, y_ref,
          deg_g_ref, deg_lg_ref, pm_ref, g_ref,
          wtx_ref, wtd_ref, wty_ref, wtl0_ref, wtl1_ref,
          wgy_ref, wgd_ref, wgx_ref, wgl0_ref, wgl1_ref,
          bias_x_ref, bias_y_ref,
          bnx_s_ref, bnx_b_ref, bny_s_ref, bny_b_ref,
          xn_ref, yn_ref,
          ybf_ref, xbf_ref, xpre_ref, acc_ref, ys1_ref, ys2_ref):
    i = pl.program_id(0)

    @pl.when(i == 0)
    def _init():
        ybf_ref[...] = y_ref[...].astype(jnp.bfloat16)
        xbf_ref[...] = x_ref[...].astype(jnp.bfloat16)
        acc_ref[...] = jnp.zeros_like(acc_ref)
        ys1_ref[...] = jnp.zeros_like(ys1_ref)
        ys2_ref[...] = jnp.zeros_like(ys2_ref)

    ybf = ybf_ref[...]
    xbf = xbf_ref[...]

    y0 = jnp.dot(mlt_ref[...].astype(jnp.bfloat16), ybf,
                 preferred_element_type=jnp.float32)
    y1 = jnp.dot(mltt_ref[...].astype(jnp.bfloat16), ybf,
                 preferred_element_type=jnp.float32)
    y_rows = y_ref[pl.ds(i * BM, BM), :]
    lane = jax.lax.broadcasted_iota(jnp.int32, (BM, N_G), 1)
    oh_pm = (lane == pm_ref[pl.ds(i * BM, BM), :]).astype(jnp.bfloat16)
    oh_g = (lane == g_ref[pl.ds(i * BM, BM), :]).astype(jnp.bfloat16)
    pmx = jnp.dot(oh_pm, xbf, preferred_element_type=jnp.float32)
    yn_rows = (_dot_t(y0.astype(jnp.bfloat16), wgl0_ref)
               + _dot_t(y1.astype(jnp.bfloat16), wgl1_ref)
               + _dot_t(y_rows.astype(jnp.bfloat16), wgy_ref)
               + _dot_t((y_rows * deg_lg_ref[pl.ds(i * BM, BM), :])
                        .astype(jnp.bfloat16), wgd_ref)
               + _dot_t(pmx.astype(jnp.bfloat16), wgx_ref)
               + bias_y_ref[...])
    yn_rows = _relu_hi(yn_rows)
    yn_ref[pl.ds(i * BM, BM), :] = yn_rows
    ys1_ref[...] += jnp.sum(yn_rows, axis=0, keepdims=True)
    ys2_ref[...] += jnp.sum(yn_rows * yn_rows, axis=0, keepdims=True)

    x0 = jnp.dot(mgt_ref[...].astype(jnp.bfloat16), xbf,
                 preferred_element_type=jnp.float32)
    x1 = jnp.dot(mgtt_ref[...].astype(jnp.bfloat16), xbf,
                 preferred_element_type=jnp.float32)
    x_rows = x_ref[pl.ds(i * XB, XB), :]
    xpre_ref[pl.ds(i * XB, XB), :] = (
        _dot_t(x0.astype(jnp.bfloat16), wtl0_ref)
        + _dot_t(x1.astype(jnp.bfloat16), wtl1_ref)
        + _dot_t(x_rows.astype(jnp.bfloat16), wtx_ref)
        + _dot_t((x_rows * deg_g_ref[pl.ds(i * XB, XB), :])
                 .astype(jnp.bfloat16), wtd_ref)
        + bias_x_ref[...])

    y_blk_bf = ybf_ref[pl.ds(i * BM, BM), :]
    acc_ref[...] += jax.lax.dot_general(
        oh_g, y_blk_bf, (((0,), (0,)), ((), ())),
        preferred_element_type=jnp.float32)

    @pl.when(i == NB - 1)
    def _finish():
        xn_pre = xpre_ref[...] + _dot_t(acc_ref[...].astype(jnp.bfloat16),
                                        wty_ref)
        xn_ref[...] = _bn(_relu_hi(xn_pre), bnx_s_ref, bnx_b_ref)
        m = ys1_ref[...] * (1.0 / N_LG)
        v = ys2_ref[...] * (1.0 / N_LG) - m * m
        scale = jax.lax.rsqrt(v + EPS) * bny_s_ref[...]
        yn_ref[...] = (yn_ref[...] - m) * scale + bny_b_ref[...]


@functools.partial(jax.jit, static_argnames=("interpret",))
def _run(x, y, deg_g, deg_lg, pm_pd2, g2,
         mask_g_t, mask_g_tt, mask_lg_t, mask_lg_tt,
         Wtx, Wtd, Wty, Wtl0, Wtl1, Wgy, Wgd, Wgx, Wgl0, Wgl1,
         bias_x, bias_y, bnx_s, bnx_b, bny_s, bny_b, interpret=False):
    const = lambda i: (0, 0)
    row_lg = lambda i: (i, 0)
    wspec = pl.BlockSpec((F, F), const)
    vspec = pl.BlockSpec((1, F), const)
    return pl.pallas_call(
        _body,
        grid=(NB,),
        in_specs=[
            pl.BlockSpec((BM, N_LG), row_lg),
            pl.BlockSpec((BM, N_LG), row_lg),
            pl.BlockSpec((XB, N_G), row_lg),
            pl.BlockSpec((XB, N_G), row_lg),
            pl.BlockSpec((N_G, F), const),
            pl.BlockSpec((N_LG, F), const),
            pl.BlockSpec((N_G, 1), const),
            pl.BlockSpec((N_LG, 1), const),
            pl.BlockSpec((N_LG, 1), const),
            pl.BlockSpec((N_LG, 1), const),
            wspec, wspec, wspec, wspec, wspec,
            wspec, wspec, wspec, wspec, wspec,
            vspec, vspec,
            vspec, vspec, vspec, vspec,
        ],
        out_specs=(pl.BlockSpec((N_G, F), const),
                   pl.BlockSpec((N_LG, F), const)),
        out_shape=(jax.ShapeDtypeStruct((N_G, F), jnp.float32),
                   jax.ShapeDtypeStruct((N_LG, F), jnp.float32)),
        scratch_shapes=[
            pltpu.VMEM((N_LG, F), jnp.bfloat16),
            pltpu.VMEM((N_G, F), jnp.bfloat16),
            pltpu.VMEM((N_G, F), jnp.float32),
            pltpu.VMEM((N_G, F), jnp.float32),
            pltpu.VMEM((1, F), jnp.float32),
            pltpu.VMEM((1, F), jnp.float32),
        ],
        compiler_params=pltpu.CompilerParams(
            dimension_semantics=("arbitrary",),
        ),
        interpret=interpret,
    )(mask_lg_t, mask_lg_tt, mask_g_t, mask_g_tt, x, y,
      deg_g, deg_lg, pm_pd2, g2,
      Wtx, Wtd, Wty, Wtl0, Wtl1, Wgy, Wgd, Wgx, Wgl0, Wgl1,
      bias_x, bias_y, bnx_s, bnx_b, bny_s, bny_b)


def kernel(g, lg, x, y, deg_g, deg_lg, pm_pd, g_t, g_tt, lg_t, lg_tt,
           mask_g_t, mask_g_tt, mask_lg_t, mask_lg_tt,
           Wtx, btx, Wtd, btd, Wty, bty, Wtl0, btl0, Wtl1, btl1,
           Wgy, bgy, Wgd, bgd, Wgx, bgx, Wgl0, bgl0, Wgl1, bgl1,
           bnx_s, bnx_b, bny_s, bny_b):
    bias_x = (btx + btd + btl0 + btl1 + bty).reshape(1, F)
    bias_y = (bgy + bgd + bgl0 + bgl1 + bgx).reshape(1, F)
    return _run(x, y, deg_g, deg_lg,
                pm_pd.astype(jnp.int32).reshape(N_LG, 1),
                g.astype(jnp.int32).reshape(N_LG, 1),
                mask_g_t, mask_g_tt, mask_lg_t, mask_lg_tt,
                Wtx, Wtd, Wty, Wtl0, Wtl1, Wgy, Wgd, Wgx, Wgl0, Wgl1,
                bias_x, bias_y,
                bnx_s.reshape(1, F), bnx_b.reshape(1, F),
                bny_s.reshape(1, F), bny_b.reshape(1, F))


# speedup vs baseline: 1.3104x; 1.1240x over previous
import functools

import jax
import jax.numpy as jnp
from jax.experimental import pallas as pl
from jax.experimental.pallas import tpu as pltpu

N_G = 1024
N_LG = 4096
F = 128
NB = 16
BM = N_LG // NB
XB = N_G // NB
HALF = F // 2
EPS = 1e-5


def _dot_t(z, w_ref):
    return jax.lax.dot_general(
        z, w_ref[...].astype(jnp.bfloat16),
        (((1,), (1,)), ((), ())), preferred_element_type=jnp.float32)


def _bn(z, s_ref, b_ref):
    m = jnp.mean(z, axis=0, keepdims=True)
    v = jnp.mean((z - m) ** 2, axis=0, keepdims=True)
    return ((z - m) * jax.lax.rsqrt(v + EPS) * s_ref[...].reshape(1, F)
            + b_ref[...].reshape(1, F))


def _relu_hi(z):
    col = jax.lax.broadcasted_iota(jnp.int32, z.shape, 1)
    return jnp.where(col < HALF, z, jnp.maximum(z, 0.0))


def _body(mlt_ref, mltt_ref, mgt_ref, mgtt_ref, x_ref, y_ref,
          deg_g_ref, deg_lg_ref, pm_ref, g_ref,
          wtx_ref, wtd_ref, wty_ref, wtl0_ref, wtl1_ref,
          wgy_ref, wgd_ref, wgx_ref, wgl0_ref, wgl1_ref,
          btx_ref, btd_ref, bty_ref, btl0_ref, btl1_ref,
          bgy_ref, bgd_ref, bgx_ref, bgl0_ref, bgl1_ref,
          bnx_s_ref, bnx_b_ref, bny_s_ref, bny_b_ref,
          xn_ref, yn_ref,
          ybf_ref, xbf_ref, xpre_ref, acc_ref, ys1_ref, ys2_ref,
          bx_ref, by_ref):
    i = pl.program_id(0)

    @pl.when(i == 0)
    def _init():
        ybf_ref[...] = y_ref[...].astype(jnp.bfloat16)
        xbf_ref[...] = x_ref[...].astype(jnp.bfloat16)
        acc_ref[...] = jnp.zeros_like(acc_ref)
        ys1_ref[...] = jnp.zeros_like(ys1_ref)
        ys2_ref[...] = jnp.zeros_like(ys2_ref)
        bx_ref[...] = (btx_ref[...] + btd_ref[...] + btl0_ref[...]
                       + btl1_ref[...] + bty_ref[...]).reshape(1, F)
        by_ref[...] = (bgy_ref[...] + bgd_ref[...] + bgl0_ref[...]
                       + bgl1_ref[...] + bgx_ref[...]).reshape(1, F)

    ybf = ybf_ref[...]
    xbf = xbf_ref[...]

    y0 = jnp.dot(mlt_ref[...].astype(jnp.bfloat16), ybf,
                 preferred_element_type=jnp.float32)
    y1 = jnp.dot(mltt_ref[...].astype(jnp.bfloat16), ybf,
                 preferred_element_type=jnp.float32)
    y_rows = y_ref[pl.ds(i * BM, BM), :]
    node = jax.lax.broadcasted_iota(jnp.int32, (N_G, BM), 0)
    ohT_pm = (node == pm_ref[pl.ds(i * BM, BM)][None, :]).astype(jnp.bfloat16)
    ohT_g = (node == g_ref[pl.ds(i * BM, BM)][None, :]).astype(jnp.bfloat16)
    pmx = jax.lax.dot_general(ohT_pm, xbf, (((0,), (0,)), ((), ())),
                              preferred_element_type=jnp.float32)
    yn_rows = (_dot_t(y0.astype(jnp.bfloat16), wgl0_ref)
               + _dot_t(y1.astype(jnp.bfloat16), wgl1_ref)
               + _dot_t(y_rows.astype(jnp.bfloat16), wgy_ref)
               + _dot_t((y_rows * deg_lg_ref[pl.ds(i * BM, BM), :])
                        .astype(jnp.bfloat16), wgd_ref)
               + _dot_t(pmx.astype(jnp.bfloat16), wgx_ref)
               + by_ref[...])
    yn_rows = _relu_hi(yn_rows)
    yn_ref[pl.ds(i * BM, BM), :] = yn_rows
    ys1_ref[...] += jnp.sum(yn_rows, axis=0, keepdims=True)
    ys2_ref[...] += jnp.sum(yn_rows * yn_rows, axis=0, keepdims=True)

    x0 = jnp.dot(mgt_ref[...].astype(jnp.bfloat16), xbf,
                 preferred_element_type=jnp.float32)
    x1 = jnp.dot(mgtt_ref[...].astype(jnp.bfloat16), xbf,
                 preferred_element_type=jnp.float32)
    x_rows = x_ref[pl.ds(i * XB, XB), :]
    xpre_ref[pl.ds(i * XB, XB), :] = (
        _dot_t(x0.astype(jnp.bfloat16), wtl0_ref)
        + _dot_t(x1.astype(jnp.bfloat16), wtl1_ref)
        + _dot_t(x_rows.astype(jnp.bfloat16), wtx_ref)
        + _dot_t((x_rows * deg_g_ref[pl.ds(i * XB, XB), :])
                 .astype(jnp.bfloat16), wtd_ref)
        + bx_ref[...])

    y_blk_bf = ybf_ref[pl.ds(i * BM, BM), :]
    acc_ref[...] += jnp.dot(ohT_g, y_blk_bf,
                            preferred_element_type=jnp.float32)

    @pl.when(i == NB - 1)
    def _finish():
        xn_pre = xpre_ref[...] + _dot_t(acc_ref[...].astype(jnp.bfloat16),
                                        wty_ref)
        xn_ref[...] = _bn(_relu_hi(xn_pre), bnx_s_ref, bnx_b_ref)
        m = ys1_ref[...] * (1.0 / N_LG)
        v = ys2_ref[...] * (1.0 / N_LG) - m * m
        scale = jax.lax.rsqrt(v + EPS) * bny_s_ref[...].reshape(1, F)
        yn_ref[...] = (yn_ref[...] - m) * scale + bny_b_ref[...].reshape(1, F)


@functools.partial(jax.jit, static_argnames=("interpret",))
def _run(x, y, deg_g, deg_lg, pm_pd, g_seg,
         mask_g_t, mask_g_tt, mask_lg_t, mask_lg_tt,
         Wtx, Wtd, Wty, Wtl0, Wtl1, Wgy, Wgd, Wgx, Wgl0, Wgl1,
         btx, btd, bty, btl0, btl1, bgy, bgd, bgx, bgl0, bgl1,
         bnx_s, bnx_b, bny_s, bny_b, interpret=False):
    const = lambda i: (0, 0)
    row_lg = lambda i: (i, 0)
    wspec = pl.BlockSpec((F, F), const)
    vspec = pl.BlockSpec((F,), lambda i: (0,))
    return pl.pallas_call(
        _body,
        grid=(NB,),
        in_specs=[
            pl.BlockSpec((BM, N_LG), row_lg),
            pl.BlockSpec((BM, N_LG), row_lg),
            pl.BlockSpec((XB, N_G), row_lg),
            pl.BlockSpec((XB, N_G), row_lg),
            pl.BlockSpec((N_G, F), const),
            pl.BlockSpec((N_LG, F), const),
            pl.BlockSpec((N_G, 1), const),
            pl.BlockSpec((N_LG, 1), const),
            pl.BlockSpec((N_LG,), lambda i: (0,)),
            pl.BlockSpec((N_LG,), lambda i: (0,)),
            wspec, wspec, wspec, wspec, wspec,
            wspec, wspec, wspec, wspec, wspec,
            vspec, vspec, vspec, vspec, vspec,
            vspec, vspec, vspec, vspec, vspec,
            vspec, vspec, vspec, vspec,
        ],
        out_specs=(pl.BlockSpec((N_G, F), const),
                   pl.BlockSpec((N_LG, F), const)),
        out_shape=(jax.ShapeDtypeStruct((N_G, F), jnp.float32),
                   jax.ShapeDtypeStruct((N_LG, F), jnp.float32)),
        scratch_shapes=[
            pltpu.VMEM((N_LG, F), jnp.bfloat16),
            pltpu.VMEM((N_G, F), jnp.bfloat16),
            pltpu.VMEM((N_G, F), jnp.float32),
            pltpu.VMEM((N_G, F), jnp.float32),
            pltpu.VMEM((1, F), jnp.float32),
            pltpu.VMEM((1, F), jnp.float32),
            pltpu.VMEM((1, F), jnp.float32),
            pltpu.VMEM((1, F), jnp.float32),
        ],
        compiler_params=pltpu.CompilerParams(
            dimension_semantics=("arbitrary",),
        ),
        interpret=interpret,
    )(mask_lg_t, mask_lg_tt, mask_g_t, mask_g_tt, x, y,
      deg_g, deg_lg, pm_pd, g_seg,
      Wtx, Wtd, Wty, Wtl0, Wtl1, Wgy, Wgd, Wgx, Wgl0, Wgl1,
      btx, btd, bty, btl0, btl1, bgy, bgd, bgx, bgl0, bgl1,
      bnx_s, bnx_b, bny_s, bny_b)


def kernel(g, lg, x, y, deg_g, deg_lg, pm_pd, g_t, g_tt, lg_t, lg_tt,
           mask_g_t, mask_g_tt, mask_lg_t, mask_lg_tt,
           Wtx, btx, Wtd, btd, Wty, bty, Wtl0, btl0, Wtl1, btl1,
           Wgy, bgy, Wgd, bgd, Wgx, bgx, Wgl0, bgl0, Wgl1, bgl1,
           bnx_s, bnx_b, bny_s, bny_b):
    return _run(x, y, deg_g, deg_lg, pm_pd, g,
                mask_g_t, mask_g_tt, mask_lg_t, mask_lg_tt,
                Wtx, Wtd, Wty, Wtl0, Wtl1, Wgy, Wgd, Wgx, Wgl0, Wgl1,
                btx, btd, bty, btl0, btl1, bgy, bgd, bgx, bgl0, bgl1,
                bnx_s, bnx_b, bny_s, bny_b)


# speedup vs baseline: 1.3130x; 1.0019x over previous
import functools

import jax
import jax.numpy as jnp
from jax.experimental import pallas as pl
from jax.experimental.pallas import tpu as pltpu

N_G = 1024
N_LG = 4096
F = 128
NB = 16
BM = N_LG // NB
XB = N_G // NB
HALF = F // 2
EPS = 1e-5


def _dot_t(z, w_ref):
    return jax.lax.dot_general(
        z, w_ref[...].astype(jnp.bfloat16),
        (((1,), (1,)), ((), ())), preferred_element_type=jnp.float32)


def _bn(z, s_ref, b_ref):
    m = jnp.mean(z, axis=0, keepdims=True)
    v = jnp.mean((z - m) ** 2, axis=0, keepdims=True)
    return ((z - m) * jax.lax.rsqrt(v + EPS) * s_ref[...].reshape(1, F)
            + b_ref[...].reshape(1, F))


def _relu_hi(z):
    col = jax.lax.broadcasted_iota(jnp.int32, z.shape, 1)
    return jnp.where(col < HALF, z, jnp.maximum(z, 0.0))


def _body(mlt_ref, mltt_ref, mgt_ref, mgtt_ref, x_ref, y_ref,
          deg_g_ref, deg_lg_ref, pm_ref, g_ref,
          wtx_ref, wtd_ref, wty_ref, wtl0_ref, wtl1_ref,
          wgy_ref, wgd_ref, wgx_ref, wgl0_ref, wgl1_ref,
          btx_ref, btd_ref, bty_ref, btl0_ref, btl1_ref,
          bgy_ref, bgd_ref, bgx_ref, bgl0_ref, bgl1_ref,
          bnx_s_ref, bnx_b_ref, bny_s_ref, bny_b_ref,
          xn_ref, yn_ref,
          ybf_ref, xbf_ref, xpre_ref, acc_ref, ys1_ref, ys2_ref,
          bx_ref, by_ref):
    i = pl.program_id(0)

    @pl.when(i == 0)
    def _init():
        ybf_ref[...] = y_ref[...].astype(jnp.bfloat16)
        xbf_ref[...] = x_ref[...].astype(jnp.bfloat16)
        acc_ref[...] = jnp.zeros_like(acc_ref)
        ys1_ref[...] = jnp.zeros_like(ys1_ref)
        ys2_ref[...] = jnp.zeros_like(ys2_ref)
        bx_ref[...] = (btx_ref[...] + btd_ref[...] + btl0_ref[...]
                       + btl1_ref[...] + bty_ref[...]).reshape(1, F)
        by_ref[...] = (bgy_ref[...] + bgd_ref[...] + bgl0_ref[...]
                       + bgl1_ref[...] + bgx_ref[...]).reshape(1, F)

    ybf = ybf_ref[...]
    xbf = xbf_ref[...]

    y0 = jnp.dot(mlt_ref[...], y_ref[...],
                 preferred_element_type=jnp.float32)
    y1 = jnp.dot(mltt_ref[...], y_ref[...],
                 preferred_element_type=jnp.float32)
    y_rows = y_ref[pl.ds(i * BM, BM), :]
    node = jax.lax.broadcasted_iota(jnp.int32, (N_G, BM), 0)
    ohT_pm = (node == pm_ref[pl.ds(i * BM, BM)][None, :]).astype(jnp.bfloat16)
    ohT_g = (node == g_ref[pl.ds(i * BM, BM)][None, :]).astype(jnp.bfloat16)
    pmx = jax.lax.dot_general(ohT_pm, xbf, (((0,), (0,)), ((), ())),
                              preferred_element_type=jnp.float32)
    yn_rows = (_dot_t(y0.astype(jnp.bfloat16), wgl0_ref)
               + _dot_t(y1.astype(jnp.bfloat16), wgl1_ref)
               + _dot_t(y_rows.astype(jnp.bfloat16), wgy_ref)
               + _dot_t((y_rows * deg_lg_ref[pl.ds(i * BM, BM), :])
                        .astype(jnp.bfloat16), wgd_ref)
               + _dot_t(pmx.astype(jnp.bfloat16), wgx_ref)
               + by_ref[...])
    yn_rows = _relu_hi(yn_rows)
    yn_ref[pl.ds(i * BM, BM), :] = yn_rows
    ys1_ref[...] += jnp.sum(yn_rows, axis=0, keepdims=True)
    ys2_ref[...] += jnp.sum(yn_rows * yn_rows, axis=0, keepdims=True)

    x0 = jnp.dot(mgt_ref[...], x_ref[...],
                 preferred_element_type=jnp.float32)
    x1 = jnp.dot(mgtt_ref[...], x_ref[...],
                 preferred_element_type=jnp.float32)
    x_rows = x_ref[pl.ds(i * XB, XB), :]
    xpre_ref[pl.ds(i * XB, XB), :] = (
        _dot_t(x0.astype(jnp.bfloat16), wtl0_ref)
        + _dot_t(x1.astype(jnp.bfloat16), wtl1_ref)
        + _dot_t(x_rows.astype(jnp.bfloat16), wtx_ref)
        + _dot_t((x_rows * deg_g_ref[pl.ds(i * XB, XB), :])
                 .astype(jnp.bfloat16), wtd_ref)
        + bx_ref[...])

    y_blk_bf = ybf_ref[pl.ds(i * BM, BM), :]
    acc_ref[...] += jnp.dot(ohT_g, y_blk_bf,
                            preferred_element_type=jnp.float32)

    @pl.when(i == NB - 1)
    def _finish():
        xn_pre = xpre_ref[...] + _dot_t(acc_ref[...].astype(jnp.bfloat16),
                                        wty_ref)
        xn_ref[...] = _bn(_relu_hi(xn_pre), bnx_s_ref, bnx_b_ref)
        m = ys1_ref[...] * (1.0 / N_LG)
        v = ys2_ref[...] * (1.0 / N_LG) - m * m
        scale = jax.lax.rsqrt(v + EPS) * bny_s_ref[...].reshape(1, F)
        yn_ref[...] = (yn_ref[...] - m) * scale + bny_b_ref[...].reshape(1, F)


@functools.partial(jax.jit, static_argnames=("interpret",))
def _run(x, y, deg_g, deg_lg, pm_pd, g_seg,
         mask_g_t, mask_g_tt, mask_lg_t, mask_lg_tt,
         Wtx, Wtd, Wty, Wtl0, Wtl1, Wgy, Wgd, Wgx, Wgl0, Wgl1,
         btx, btd, bty, btl0, btl1, bgy, bgd, bgx, bgl0, bgl1,
         bnx_s, bnx_b, bny_s, bny_b, interpret=False):
    const = lambda i: (0, 0)
    row_lg = lambda i: (i, 0)
    wspec = pl.BlockSpec((F, F), const)
    vspec = pl.BlockSpec((F,), lambda i: (0,))
    return pl.pallas_call(
        _body,
        grid=(NB,),
        in_specs=[
            pl.BlockSpec((BM, N_LG), row_lg),
            pl.BlockSpec((BM, N_LG), row_lg),
            pl.BlockSpec((XB, N_G), row_lg),
            pl.BlockSpec((XB, N_G), row_lg),
            pl.BlockSpec((N_G, F), const),
            pl.BlockSpec((N_LG, F), const),
            pl.BlockSpec((N_G, 1), const),
            pl.BlockSpec((N_LG, 1), const),
            pl.BlockSpec((N_LG,), lambda i: (0,)),
            pl.BlockSpec((N_LG,), lambda i: (0,)),
            wspec, wspec, wspec, wspec, wspec,
            wspec, wspec, wspec, wspec, wspec,
            vspec, vspec, vspec, vspec, vspec,
            vspec, vspec, vspec, vspec, vspec,
            vspec, vspec, vspec, vspec,
        ],
        out_specs=(pl.BlockSpec((N_G, F), const),
                   pl.BlockSpec((N_LG, F), const)),
        out_shape=(jax.ShapeDtypeStruct((N_G, F), jnp.float32),
                   jax.ShapeDtypeStruct((N_LG, F), jnp.float32)),
        scratch_shapes=[
            pltpu.VMEM((N_LG, F), jnp.bfloat16),
            pltpu.VMEM((N_G, F), jnp.bfloat16),
            pltpu.VMEM((N_G, F), jnp.float32),
            pltpu.VMEM((N_G, F), jnp.float32),
            pltpu.VMEM((1, F), jnp.float32),
            pltpu.VMEM((1, F), jnp.float32),
            pltpu.VMEM((1, F), jnp.float32),
            pltpu.VMEM((1, F), jnp.float32),
        ],
        compiler_params=pltpu.CompilerParams(
            dimension_semantics=("arbitrary",),
        ),
        interpret=interpret,
    )(mask_lg_t, mask_lg_tt, mask_g_t, mask_g_tt, x, y,
      deg_g, deg_lg, pm_pd, g_seg,
      Wtx, Wtd, Wty, Wtl0, Wtl1, Wgy, Wgd, Wgx, Wgl0, Wgl1,
      btx, btd, bty, btl0, btl1, bgy, bgd, bgx, bgl0, bgl1,
      bnx_s, bnx_b, bny_s, bny_b)


def kernel(g, lg, x, y, deg_g, deg_lg, pm_pd, g_t, g_tt, lg_t, lg_tt,
           mask_g_t, mask_g_tt, mask_lg_t, mask_lg_tt,
           Wtx, btx, Wtd, btd, Wty, bty, Wtl0, btl0, Wtl1, btl1,
           Wgy, bgy, Wgd, bgd, Wgx, bgx, Wgl0, bgl0, Wgl1, bgl1,
           bnx_s, bnx_b, bny_s, bny_b):
    return _run(x, y, deg_g, deg_lg, pm_pd, g,
                mask_g_t, mask_g_tt, mask_lg_t, mask_lg_tt,
                Wtx, Wtd, Wty, Wtl0, Wtl1, Wgy, Wgd, Wgx, Wgl0, Wgl1,
                btx, btd, bty, btl0, btl1, bgy, bgd, bgx, bgl0, bgl1,
                bnx_s, bnx_b, bny_s, bny_b)
